# baseline (speedup 1.0000x reference)
import functools

import jax
import jax.numpy as jnp
from jax import lax
from jax.experimental import pallas as pl
from jax.experimental.pallas import tpu as pltpu
from jax.experimental.pallas import tpu_sc as plsc

N = 4096
TM = 512
N_TC = 3072
N_SC = N - N_TC
NW = 32
RPW = N_SC // NW
GRP = 8
NEG_INF = float("-inf")


def _tc_body(prev_ref, nblk_ref, ntT_ref, a_ref, wphi_ref, wth_ref, out_ref):
    i = pl.program_id(0)
    w0 = wth_ref[0, 0]
    w1 = wth_ref[1, 0]
    w2 = wth_ref[2, 0]
    c0 = w0 * w0
    c1 = w1 * w1
    c2 = w2 * w2

    x0 = ntT_ref[0:1, :]
    x1 = ntT_ref[1:2, :]
    x2 = ntT_ref[2:3, :]
    g0 = x0 * (-2.0 * c0)
    g1 = x1 * (-2.0 * c1)
    g2 = x2 * (-2.0 * c2)
    sq = x0 * x0 * c0 + x1 * x1 * c1 + x2 * x2 * c2

    y0 = nblk_ref[:, 0:1]
    y1 = nblk_ref[:, 1:2]
    y2 = nblk_ref[:, 2:3]

    z = ((sq + y0 * g0) + y1 * g1) + y2 * g2

    mask = a_ref[:, :] != 0
    m = jnp.max(jnp.where(mask, z, NEG_INF), axis=1, keepdims=True)
    mrow = m.T
    xi0 = ntT_ref[0:1, pl.ds(i * TM, TM)]
    xi1 = ntT_ref[1:2, pl.ds(i * TM, TM)]
    xi2 = ntT_ref[2:3, pl.ds(i * TM, TM)]
    sqi = xi0 * xi0 * c0 + xi1 * xi1 * c1 + xi2 * xi2 * c2
    d2 = sqi + mrow
    maxd = jnp.where(mrow == NEG_INF, NEG_INF, jnp.sqrt(jnp.maximum(d2, 0.0)))

    half_wmean = 0.5 * jnp.mean(wphi_ref[0, :])
    out_ref[0:1, :] = 0.5 * prev_ref[0:1, :] + maxd * half_wmean


def _tc_part(prev, nodes, ntT, adjacency, wphi, wth):
    grid = (N_TC // TM,)
    return pl.pallas_call(
        _tc_body,
        grid=grid,
        in_specs=[
            pl.BlockSpec((1, TM), lambda i: (0, i)),
            pl.BlockSpec((TM, 3), lambda i: (i, 0)),
            pl.BlockSpec((3, N), lambda i: (0, 0)),
            pl.BlockSpec((TM, N), lambda i: (i, 0)),
            pl.BlockSpec((1, wphi.shape[1]), lambda i: (0, 0)),
            pl.BlockSpec((3, 1), lambda i: (0, 0)),
        ],
        out_specs=pl.BlockSpec((1, TM), lambda i: (0, i)),
        out_shape=jax.ShapeDtypeStruct((1, N_TC), jnp.float32),
    )(prev, nodes, ntT, adjacency, wphi, wth)


def _sc_body(adj_hbm, ntT_hbm, prev_hbm, wphi_hbm, wth_hbm, out_hbm,
             xs_v, a_v, prev_v, out_v, wphi_v, wth_v):
    wid = lax.axis_index("c") * 16 + lax.axis_index("s")
    base = N_TC + wid * RPW
    lane = lax.iota(jnp.int32, 16)

    pltpu.sync_copy(ntT_hbm, xs_v)
    pltpu.sync_copy(prev_hbm.at[pl.ds(base, RPW)], prev_v)
    pltpu.sync_copy(wphi_hbm, wphi_v)
    pltpu.sync_copy(wth_hbm, wth_v)

    def _take(vec, idx):
        dn = lax.GatherDimensionNumbers(
            offset_dims=(), collapsed_slice_dims=(0,), start_index_map=(0,))
        return lax.gather(vec, idx[:, None], dn, (1,),
                          mode=lax.GatherScatterMode.PROMISE_IN_BOUNDS)

    def _bcast_lane(vec, pos):
        return _take(vec, jnp.full((16,), pos, jnp.int32))

    def _bcast_max(vec):
        for k in (8, 4, 2, 1):
            vec = jnp.maximum(vec, _take(vec, lane ^ k))
        return vec

    def _bcast_sum(vec):
        for k in (8, 4, 2, 1):
            vec = vec + _take(vec, lane ^ k)
        return vec

    wv = wth_v[:]
    w0 = _bcast_lane(wv, 0)
    w1 = _bcast_lane(wv, 1)
    w2 = _bcast_lane(wv, 2)
    c0 = w0 * w0
    c1 = w1 * w1
    c2 = w2 * w2

    def _wsum(i, acc):
        return acc + wphi_v[pl.ds(i * 16, 16)]
    wacc = lax.fori_loop(0, 16, _wsum, jnp.zeros((16,), jnp.float32))
    half_wmean = _bcast_sum(wacc) * jnp.float32(0.5 / 256.0)

    outvec = jnp.zeros((16,), jnp.float32)
    for g in range(RPW // GRP):
        row0 = base + g * GRP
        pltpu.sync_copy(adj_hbm.at[pl.ds(row0, GRP), :], a_v)

        ys = []
        for r in range(GRP):
            lr = g * GRP + r
            pos = lr % 16
            cs = base + (lr // 16) * 16
            y0 = _bcast_lane(xs_v[0, pl.ds(cs, 16)], pos)
            y1 = _bcast_lane(xs_v[1, pl.ds(cs, 16)], pos)
            y2 = _bcast_lane(xs_v[2, pl.ds(cs, 16)], pos)
            ys.append((y0 * (-2.0), y1 * (-2.0), y2 * (-2.0),
                       y0 * y0 * c0 + y1 * y1 * c1 + y2 * y2 * c2))

        def _chunk(cidx, accs):
            sl = pl.ds(cidx * 16, 16)
            x0c = xs_v[0, sl]
            x1c = xs_v[1, sl]
            x2c = xs_v[2, sl]
            t0 = x0c * c0
            t1 = x1c * c1
            t2 = x2c * c2
            sqc = t0 * x0c + t1 * x1c + t2 * x2c
            new = []
            for r in range(GRP):
                y0n, y1n, y2n, _ = ys[r]
                z = ((sqc + t0 * y0n) + t1 * y1n) + t2 * y2n
                va = a_v[r, sl]
                new.append(
                    jnp.maximum(accs[r], jnp.where(va != 0, z, NEG_INF)))
            return tuple(new)

        accs = lax.fori_loop(
            0, N // 16, _chunk,
            tuple(jnp.full((16,), NEG_INF, jnp.float32)
                  for _ in range(GRP)))

        for r in range(GRP):
            lr = g * GRP + r
            _, _, _, sqi = ys[r]
            m = _bcast_max(accs[r])
            d2 = m + sqi
            dd = jnp.where(d2 > 0.0, d2, 1.0)
            s = (dd + 1.0) * 0.5
            for _ in range(20):
                s = (s + dd / s) * 0.5
            maxd = jnp.where(d2 == NEG_INF, NEG_INF,
                             jnp.where(d2 > 0.0, s, 0.0))
            pc = prev_v[pl.ds((lr // 16) * 16, 16)]
            prv = _bcast_lane(pc, lr % 16)
            res = prv * 0.5 + maxd * half_wmean
            outvec = jnp.where(lane == (lr % 16), res, outvec)
            if lr % 16 == 15:
                out_v[pl.ds(lr - 15, 16)] = outvec
                outvec = jnp.zeros((16,), jnp.float32)

    pltpu.sync_copy(out_v, out_hbm.at[pl.ds(wid * RPW, RPW)])


def _sc_part(prev, ntT, adjacency, wphi, wth):
    mesh = plsc.VectorSubcoreMesh(core_axis_name="c", subcore_axis_name="s")
    fn = functools.partial(
        pl.kernel,
        mesh=mesh,
        out_type=jax.ShapeDtypeStruct((N_SC,), jnp.float32),
        scratch_types=[
            pltpu.VMEM((3, N), jnp.float32),
            pltpu.VMEM((GRP, N), jnp.int32),
            pltpu.VMEM((RPW,), jnp.float32),
            pltpu.VMEM((RPW,), jnp.float32),
            pltpu.VMEM((256,), jnp.float32),
            pltpu.VMEM((16,), jnp.float32),
        ],
    )(_sc_body)
    return fn(adjacency, ntT, prev, wphi, wth)


@jax.jit
def _run(prev, nodes, adjacency, wphi, wth):
    prev2 = prev.reshape(1, N)
    wphi2 = wphi.reshape(1, -1)
    ntT = nodes.T
    tc = _tc_part(prev2, nodes, ntT, adjacency, wphi2, wth)
    wth16 = jnp.pad(wth.reshape(3), (0, 13))
    sc = _sc_part(prev, ntT, adjacency, wphi, wth16)
    return jnp.concatenate([tc.reshape(N_TC), sc])


def kernel(previous_inclusion_score, nodes, adjacency_matrix, W_phi, W_theta):
    return _run(previous_inclusion_score, nodes, adjacency_matrix, W_phi,
                W_theta)


# speedup vs baseline: 1.3421x; 1.3421x over previous
import functools

import jax
import jax.numpy as jnp
from jax import lax
from jax.experimental import pallas as pl
from jax.experimental.pallas import tpu as pltpu
from jax.experimental.pallas import tpu_sc as plsc

N = 4096
TM = 512
N_TC = 3584
N_SC = N - N_TC
NW = 32
RPW = N_SC // NW
GRP = 8
NEG_INF = float("-inf")


def _tc_body(prev_ref, nblk_ref, ntT_ref, a_ref, wphi_ref, wth_ref, out_ref):
    i = pl.program_id(0)
    w0 = wth_ref[0, 0]
    w1 = wth_ref[1, 0]
    w2 = wth_ref[2, 0]
    c0 = w0 * w0
    c1 = w1 * w1
    c2 = w2 * w2

    x0 = ntT_ref[0:1, :]
    x1 = ntT_ref[1:2, :]
    x2 = ntT_ref[2:3, :]
    g0 = x0 * (-2.0 * c0)
    g1 = x1 * (-2.0 * c1)
    g2 = x2 * (-2.0 * c2)
    sq = x0 * x0 * c0 + x1 * x1 * c1 + x2 * x2 * c2

    y0 = nblk_ref[:, 0:1]
    y1 = nblk_ref[:, 1:2]
    y2 = nblk_ref[:, 2:3]

    z = ((sq + y0 * g0) + y1 * g1) + y2 * g2

    mask = a_ref[:, :] != 0
    m = jnp.max(jnp.where(mask, z, NEG_INF), axis=1, keepdims=True)
    mrow = m.T
    xi0 = ntT_ref[0:1, pl.ds(i * TM, TM)]
    xi1 = ntT_ref[1:2, pl.ds(i * TM, TM)]
    xi2 = ntT_ref[2:3, pl.ds(i * TM, TM)]
    sqi = xi0 * xi0 * c0 + xi1 * xi1 * c1 + xi2 * xi2 * c2
    d2 = sqi + mrow
    maxd = jnp.where(mrow == NEG_INF, NEG_INF, jnp.sqrt(jnp.maximum(d2, 0.0)))

    half_wmean = 0.5 * jnp.mean(wphi_ref[0, :])
    out_ref[0:1, :] = 0.5 * prev_ref[0:1, :] + maxd * half_wmean


def _tc_part(prev, nodes, ntT, adjacency, wphi, wth):
    grid = (N_TC // TM,)
    return pl.pallas_call(
        _tc_body,
        grid=grid,
        in_specs=[
            pl.BlockSpec((1, TM), lambda i: (0, i)),
            pl.BlockSpec((TM, 3), lambda i: (i, 0)),
            pl.BlockSpec((3, N), lambda i: (0, 0)),
            pl.BlockSpec((TM, N), lambda i: (i, 0)),
            pl.BlockSpec((1, wphi.shape[1]), lambda i: (0, 0)),
            pl.BlockSpec((3, 1), lambda i: (0, 0)),
        ],
        out_specs=pl.BlockSpec((1, TM), lambda i: (0, i)),
        out_shape=jax.ShapeDtypeStruct((1, N_TC), jnp.float32),
    )(prev, nodes, ntT, adjacency, wphi, wth)


def _sc_body(adj_hbm, ntT_hbm, prev_hbm, wphi_hbm, wth_hbm, out_hbm,
             xs_v, ts_v, a_v, prev_v, out_v, wphi_v, wth_v):
    wid = lax.axis_index("c") * 16 + lax.axis_index("s")
    base = N_TC + wid * RPW
    lane = lax.iota(jnp.int32, 16)

    pltpu.sync_copy(ntT_hbm, xs_v)
    pltpu.sync_copy(prev_hbm.at[pl.ds(base, RPW)], prev_v)
    pltpu.sync_copy(wphi_hbm, wphi_v)
    pltpu.sync_copy(wth_hbm, wth_v)

    def _take(vec, idx):
        dn = lax.GatherDimensionNumbers(
            offset_dims=(), collapsed_slice_dims=(0,), start_index_map=(0,))
        return lax.gather(vec, idx[:, None], dn, (1,),
                          mode=lax.GatherScatterMode.PROMISE_IN_BOUNDS)

    def _bcast_lane(vec, pos):
        return _take(vec, jnp.full((16,), pos, jnp.int32))

    def _bcast_max(vec):
        for k in (8, 4, 2, 1):
            vec = jnp.maximum(vec, _take(vec, lane ^ k))
        return vec

    def _bcast_sum(vec):
        for k in (8, 4, 2, 1):
            vec = vec + _take(vec, lane ^ k)
        return vec

    wv = wth_v[:]
    w0 = _bcast_lane(wv, 0)
    w1 = _bcast_lane(wv, 1)
    w2 = _bcast_lane(wv, 2)
    c0 = w0 * w0
    c1 = w1 * w1
    c2 = w2 * w2

    def _wsum(i, acc):
        return acc + wphi_v[pl.ds(i * 16, 16)]
    wacc = lax.fori_loop(0, 16, _wsum, jnp.zeros((16,), jnp.float32))
    half_wmean = _bcast_sum(wacc) * jnp.float32(0.5 / 256.0)

    def _pre(cidx, carry):
        sl = pl.ds(cidx * 16, 16)
        x0c = xs_v[0, sl]
        x1c = xs_v[1, sl]
        x2c = xs_v[2, sl]
        t0 = x0c * c0
        t1 = x1c * c1
        t2 = x2c * c2
        ts_v[0, sl] = t0
        ts_v[1, sl] = t1
        ts_v[2, sl] = t2
        ts_v[3, sl] = t0 * x0c + t1 * x1c + t2 * x2c
        return carry
    lax.fori_loop(0, N // 16, _pre, jnp.int32(0))

    outvec = jnp.zeros((16,), jnp.float32)
    for g in range(RPW // GRP):
        row0 = base + g * GRP
        pltpu.sync_copy(adj_hbm.at[pl.ds(row0, GRP), :], a_v)

        ys = []
        for r in range(GRP):
            lr = g * GRP + r
            pos = lr % 16
            cs = base + (lr // 16) * 16
            y0 = _bcast_lane(xs_v[0, pl.ds(cs, 16)], pos)
            y1 = _bcast_lane(xs_v[1, pl.ds(cs, 16)], pos)
            y2 = _bcast_lane(xs_v[2, pl.ds(cs, 16)], pos)
            ys.append((y0 * (-2.0), y1 * (-2.0), y2 * (-2.0),
                       y0 * y0 * c0 + y1 * y1 * c1 + y2 * y2 * c2))

        def _chunk(cidx, accs):
            sl = pl.ds(cidx * 16, 16)
            t0 = ts_v[0, sl]
            t1 = ts_v[1, sl]
            t2 = ts_v[2, sl]
            sqc = ts_v[3, sl]
            new = []
            for r in range(GRP):
                y0n, y1n, y2n, _ = ys[r]
                z = ((sqc + t0 * y0n) + t1 * y1n) + t2 * y2n
                va = a_v[r, sl]
                new.append(
                    jnp.maximum(accs[r], jnp.where(va != 0, z, NEG_INF)))
            return tuple(new)

        accs = lax.fori_loop(
            0, N // 16, _chunk,
            tuple(jnp.full((16,), NEG_INF, jnp.float32)
                  for _ in range(GRP)))

        for r in range(GRP):
            lr = g * GRP + r
            _, _, _, sqi = ys[r]
            m = _bcast_max(accs[r])
            d2 = m + sqi
            dd = jnp.where(d2 > 0.0, d2, 1.0)
            s = (dd + 1.0) * 0.5
            for _ in range(20):
                s = (s + dd / s) * 0.5
            maxd = jnp.where(d2 == NEG_INF, NEG_INF,
                             jnp.where(d2 > 0.0, s, 0.0))
            pc = prev_v[pl.ds((lr // 16) * 16, 16)]
            prv = _bcast_lane(pc, lr % 16)
            res = prv * 0.5 + maxd * half_wmean
            outvec = jnp.where(lane == (lr % 16), res, outvec)
            if lr % 16 == 15:
                out_v[pl.ds(lr - 15, 16)] = outvec
                outvec = jnp.zeros((16,), jnp.float32)

    pltpu.sync_copy(out_v, out_hbm.at[pl.ds(wid * RPW, RPW)])


def _sc_part(prev, ntT, adjacency, wphi, wth):
    mesh = plsc.VectorSubcoreMesh(core_axis_name="c", subcore_axis_name="s")
    fn = functools.partial(
        pl.kernel,
        mesh=mesh,
        out_type=jax.ShapeDtypeStruct((N_SC,), jnp.float32),
        scratch_types=[
            pltpu.VMEM((3, N), jnp.float32),
            pltpu.VMEM((4, N), jnp.float32),
            pltpu.VMEM((GRP, N), jnp.int32),
            pltpu.VMEM((RPW,), jnp.float32),
            pltpu.VMEM((RPW,), jnp.float32),
            pltpu.VMEM((256,), jnp.float32),
            pltpu.VMEM((16,), jnp.float32),
        ],
    )(_sc_body)
    return fn(adjacency, ntT, prev, wphi, wth)


@jax.jit
def _run(prev, nodes, adjacency, wphi, wth):
    prev2 = prev.reshape(1, N)
    wphi2 = wphi.reshape(1, -1)
    ntT = nodes.T
    wth16 = jnp.pad(wth.reshape(3), (0, 13))
    sc = _sc_part(prev, ntT, adjacency, wphi, wth16)
    tc = _tc_part(prev2, nodes, ntT, adjacency, wphi2, wth)
    return jnp.concatenate([tc.reshape(N_TC), sc])


def kernel(previous_inclusion_score, nodes, adjacency_matrix, W_phi, W_theta):
    return _run(previous_inclusion_score, nodes, adjacency_matrix, W_phi,
                W_theta)


# speedup vs baseline: 1.9057x; 1.4200x over previous
import jax
import jax.numpy as jnp
from jax.experimental import pallas as pl

N = 4096
TM = 256


def _body(prev_ref, nblk_ref, ntT_ref, a_ref, wphi_ref, wth_ref, out_ref):
    i = pl.program_id(0)
    w0 = wth_ref[0, 0]
    w1 = wth_ref[1, 0]
    w2 = wth_ref[2, 0]
    c0 = w0 * w0
    c1 = w1 * w1
    c2 = w2 * w2

    x0 = ntT_ref[0:1, :]
    x1 = ntT_ref[1:2, :]
    x2 = ntT_ref[2:3, :]
    g0 = x0 * (-2.0 * c0)
    g1 = x1 * (-2.0 * c1)
    g2 = x2 * (-2.0 * c2)
    sq = x0 * x0 * c0 + x1 * x1 * c1 + x2 * x2 * c2

    y0 = nblk_ref[:, 0:1]
    y1 = nblk_ref[:, 1:2]
    y2 = nblk_ref[:, 2:3]

    z = ((sq + y0 * g0) + y1 * g1) + y2 * g2

    mask = a_ref[:, :] != 0
    neg = jnp.float32(-jnp.inf)
    m = jnp.max(jnp.where(mask, z, neg), axis=1, keepdims=True)
    mrow = m.T
    xi0 = ntT_ref[0:1, pl.ds(i * TM, TM)]
    xi1 = ntT_ref[1:2, pl.ds(i * TM, TM)]
    xi2 = ntT_ref[2:3, pl.ds(i * TM, TM)]
    sqi = xi0 * xi0 * c0 + xi1 * xi1 * c1 + xi2 * xi2 * c2
    d2 = sqi + mrow
    maxd = jnp.where(mrow == neg, neg, jnp.sqrt(jnp.maximum(d2, 0.0)))

    half_wmean = 0.5 * jnp.mean(wphi_ref[0, :])
    out_ref[0:1, :] = 0.5 * prev_ref[0:1, :] + maxd * half_wmean


@jax.jit
def _run(prev, nodes, adjacency, wphi, wth):
    prev = prev.reshape(1, N)
    wphi = wphi.reshape(1, -1)
    ntT = nodes.T
    grid = (N // TM,)
    out = pl.pallas_call(
        _body,
        grid=grid,
        in_specs=[
            pl.BlockSpec((1, TM), lambda i: (0, i)),
            pl.BlockSpec((TM, 3), lambda i: (i, 0)),
            pl.BlockSpec((3, N), lambda i: (0, 0)),
            pl.BlockSpec((TM, N), lambda i: (i, 0)),
            pl.BlockSpec((1, wphi.shape[1]), lambda i: (0, 0)),
            pl.BlockSpec((3, 1), lambda i: (0, 0)),
        ],
        out_specs=pl.BlockSpec((1, TM), lambda i: (0, i)),
        out_shape=jax.ShapeDtypeStruct((1, N), jnp.float32),
    )(prev, nodes, ntT, adjacency, wphi, wth)
    return out.reshape(N)


def kernel(previous_inclusion_score, nodes, adjacency_matrix, W_phi, W_theta):
    return _run(previous_inclusion_score, nodes, adjacency_matrix, W_phi,
                W_theta)


# speedup vs baseline: 2.0911x; 1.0972x over previous
import jax
import jax.numpy as jnp
from jax.experimental import pallas as pl

N = 4096
TM = 1024


def _body(prev_ref, nblk_ref, ntT_ref, a_ref, wphi_ref, wth_ref, out_ref):
    i = pl.program_id(0)
    w0 = wth_ref[0, 0]
    w1 = wth_ref[1, 0]
    w2 = wth_ref[2, 0]
    c0 = w0 * w0
    c1 = w1 * w1
    c2 = w2 * w2

    x0 = ntT_ref[0:1, :]
    x1 = ntT_ref[1:2, :]
    x2 = ntT_ref[2:3, :]
    g0 = x0 * (-2.0 * c0)
    g1 = x1 * (-2.0 * c1)
    g2 = x2 * (-2.0 * c2)
    sq = x0 * x0 * c0 + x1 * x1 * c1 + x2 * x2 * c2

    y0 = nblk_ref[:, 0:1]
    y1 = nblk_ref[:, 1:2]
    y2 = nblk_ref[:, 2:3]

    z = ((sq + y0 * g0) + y1 * g1) + y2 * g2

    mask = a_ref[:, :] != 0
    neg = jnp.float32(-jnp.inf)
    m = jnp.max(jnp.where(mask, z, neg), axis=1, keepdims=True)
    mrow = m.T
    xi0 = ntT_ref[0:1, pl.ds(i * TM, TM)]
    xi1 = ntT_ref[1:2, pl.ds(i * TM, TM)]
    xi2 = ntT_ref[2:3, pl.ds(i * TM, TM)]
    sqi = xi0 * xi0 * c0 + xi1 * xi1 * c1 + xi2 * xi2 * c2
    d2 = sqi + mrow
    maxd = jnp.where(mrow == neg, neg, jnp.sqrt(jnp.maximum(d2, 0.0)))

    half_wmean = 0.5 * jnp.mean(wphi_ref[0, :])
    out_ref[0:1, :] = 0.5 * prev_ref[0:1, :] + maxd * half_wmean


@jax.jit
def _run(prev, nodes, adjacency, wphi, wth):
    prev = prev.reshape(1, N)
    wphi = wphi.reshape(1, -1)
    ntT = nodes.T
    grid = (N // TM,)
    out = pl.pallas_call(
        _body,
        grid=grid,
        in_specs=[
            pl.BlockSpec((1, TM), lambda i: (0, i)),
            pl.BlockSpec((TM, 3), lambda i: (i, 0)),
            pl.BlockSpec((3, N), lambda i: (0, 0)),
            pl.BlockSpec((TM, N), lambda i: (i, 0)),
            pl.BlockSpec((1, wphi.shape[1]), lambda i: (0, 0)),
            pl.BlockSpec((3, 1), lambda i: (0, 0)),
        ],
        out_specs=pl.BlockSpec((1, TM), lambda i: (0, i)),
        out_shape=jax.ShapeDtypeStruct((1, N), jnp.float32),
    )(prev, nodes, ntT, adjacency, wphi, wth)
    return out.reshape(N)


def kernel(previous_inclusion_score, nodes, adjacency_matrix, W_phi, W_theta):
    return _run(previous_inclusion_score, nodes, adjacency_matrix, W_phi,
                W_theta)


# speedup vs baseline: 2.0967x; 1.0027x over previous
import jax
import jax.numpy as jnp
from jax.experimental import pallas as pl

N = 4096
TM = 512


def _body(prev_ref, nblk_ref, ntT_ref, a_ref, wphi_ref, wth_ref, out_ref):
    i = pl.program_id(0)
    w0 = wth_ref[0, 0]
    w1 = wth_ref[1, 0]
    w2 = wth_ref[2, 0]
    c0 = w0 * w0
    c1 = w1 * w1
    c2 = w2 * w2

    x0 = ntT_ref[0:1, :]
    x1 = ntT_ref[1:2, :]
    x2 = ntT_ref[2:3, :]
    g0 = x0 * (-2.0 * c0)
    g1 = x1 * (-2.0 * c1)
    g2 = x2 * (-2.0 * c2)
    sq = x0 * x0 * c0 + x1 * x1 * c1 + x2 * x2 * c2

    y0 = nblk_ref[:, 0:1]
    y1 = nblk_ref[:, 1:2]
    y2 = nblk_ref[:, 2:3]

    z = ((sq + y0 * g0) + y1 * g1) + y2 * g2

    mask = a_ref[:, :] != 0
    neg = jnp.float32(-jnp.inf)
    m = jnp.max(jnp.where(mask, z, neg), axis=1, keepdims=True)
    mrow = m.T
    xi0 = ntT_ref[0:1, pl.ds(i * TM, TM)]
    xi1 = ntT_ref[1:2, pl.ds(i * TM, TM)]
    xi2 = ntT_ref[2:3, pl.ds(i * TM, TM)]
    sqi = xi0 * xi0 * c0 + xi1 * xi1 * c1 + xi2 * xi2 * c2
    d2 = sqi + mrow
    maxd = jnp.where(mrow == neg, neg, jnp.sqrt(jnp.maximum(d2, 0.0)))

    half_wmean = 0.5 * jnp.mean(wphi_ref[0, :])
    out_ref[0:1, :] = 0.5 * prev_ref[0:1, :] + maxd * half_wmean


@jax.jit
def _run(prev, nodes, adjacency, wphi, wth):
    prev = prev.reshape(1, N)
    wphi = wphi.reshape(1, -1)
    ntT = nodes.T
    grid = (N // TM,)
    out = pl.pallas_call(
        _body,
        grid=grid,
        in_specs=[
            pl.BlockSpec((1, TM), lambda i: (0, i)),
            pl.BlockSpec((TM, 3), lambda i: (i, 0)),
            pl.BlockSpec((3, N), lambda i: (0, 0)),
            pl.BlockSpec((TM, N), lambda i: (i, 0)),
            pl.BlockSpec((1, wphi.shape[1]), lambda i: (0, 0)),
            pl.BlockSpec((3, 1), lambda i: (0, 0)),
        ],
        out_specs=pl.BlockSpec((1, TM), lambda i: (0, i)),
        out_shape=jax.ShapeDtypeStruct((1, N), jnp.float32),
    )(prev, nodes, ntT, adjacency, wphi, wth)
    return out.reshape(N)


def kernel(previous_inclusion_score, nodes, adjacency_matrix, W_phi, W_theta):
    return _run(previous_inclusion_score, nodes, adjacency_matrix, W_phi,
                W_theta)
